# initial kernel scaffold (unmeasured)
import jax
import jax.numpy as jnp
from jax import lax
from jax.experimental import pallas as pl
from jax.experimental.pallas import tpu as pltpu

N_DEV = 8
AXIS = "i"


def kernel(x, w_mat):
    partial = jnp.dot(x, w_mat, preferred_element_type=jnp.float32)
    return _ring_allreduce(partial)


def _ring_allreduce(y):
    m, n = y.shape
    ch = m // N_DEV

    def body(y_hbm, out_hbm, acc, recv, loc, send_sems, recv_sems, dma_sem,
             credit_sem):
        me = lax.axis_index(AXIS)
        left = lax.rem(me + N_DEV - 1, N_DEV)
        right = lax.rem(me + 1, N_DEV)

        barrier = pltpu.get_barrier_semaphore()
        for nbr in (left, right):
            pl.semaphore_signal(barrier, inc=1, device_id=(nbr,),
                                device_id_type=pl.DeviceIdType.MESH)
        pl.semaphore_wait(barrier, 2)

        cp = pltpu.make_async_copy(y_hbm.at[pl.ds(me * ch, ch), :], acc,
                                   dma_sem)
        cp.start()
        cp.wait()

        n_steps = 2 * (N_DEV - 1)
        for k in range(n_steps):
            slot = k % 2
            is_rs = k < N_DEV - 1

            if k >= 2:
                pl.semaphore_wait(credit_sem, 1)

            rdma = pltpu.make_async_remote_copy(
                src_ref=acc,
                dst_ref=recv.at[slot],
                send_sem=send_sems.at[slot],
                recv_sem=recv_sems.at[slot],
                device_id=(right,),
                device_id_type=pl.DeviceIdType.MESH,
            )
            rdma.start()

            if is_rs:
                c_in = lax.rem(me - (k + 1) + 2 * N_DEV, N_DEV)
                cpl = pltpu.make_async_copy(
                    y_hbm.at[pl.ds(c_in * ch, ch), :], loc, dma_sem)
                cpl.start()
                cpl.wait()

            rdma.wait()

            if is_rs:
                acc[...] = recv[slot] + loc[...]
            else:
                acc[...] = recv[slot]

            if k <= n_steps - 3:
                pl.semaphore_signal(credit_sem, inc=1, device_id=(left,),
                                    device_id_type=pl.DeviceIdType.MESH)

            if k == N_DEV - 2:
                c_own = lax.rem(me + 1, N_DEV)
                st = pltpu.make_async_copy(
                    acc, out_hbm.at[pl.ds(c_own * ch, ch), :], dma_sem)
                st.start()
                st.wait()
            if not is_rs:
                t = k - (N_DEV - 1)
                c_recv = lax.rem(me - t + 2 * N_DEV, N_DEV)
                st = pltpu.make_async_copy(
                    acc, out_hbm.at[pl.ds(c_recv * ch, ch), :], dma_sem)
                st.start()
                st.wait()

    return pl.pallas_call(
        body,
        out_shape=jax.ShapeDtypeStruct((m, n), jnp.float32),
        in_specs=[pl.BlockSpec(memory_space=pltpu.ANY)],
        out_specs=pl.BlockSpec(memory_space=pltpu.ANY),
        scratch_shapes=[
            pltpu.VMEM((ch, n), jnp.float32),
            pltpu.VMEM((2, ch, n), jnp.float32),
            pltpu.VMEM((ch, n), jnp.float32),
            pltpu.SemaphoreType.DMA((2,)),
            pltpu.SemaphoreType.DMA((2,)),
            pltpu.SemaphoreType.DMA,
            pltpu.SemaphoreType.REGULAR,
        ],
        compiler_params=pltpu.CompilerParams(collective_id=0),
    )(y)


# baseline (device time: 2816143 ns/iter reference)
import jax
import jax.numpy as jnp
from jax import lax
from jax.experimental import pallas as pl
from jax.experimental.pallas import tpu as pltpu

N_DEV = 8
N_PASS = 2
AXIS = "i"


def kernel(x, w_mat):
    partial = jnp.dot(x, w_mat, preferred_element_type=jnp.float32)
    return _ring_allreduce(partial)


def _ring_allreduce(y):
    m, n = y.shape
    ch = m // N_DEV // N_PASS

    def body(y_hbm, out_hbm, acc, recv, loc, send_sems, recv_sems, dma_sem,
             credit_sem):
        me = lax.axis_index(AXIS)
        left = lax.rem(me + N_DEV - 1, N_DEV)
        right = lax.rem(me + 1, N_DEV)

        barrier = pltpu.get_barrier_semaphore()
        for nbr in (left, right):
            pl.semaphore_signal(barrier, inc=1, device_id=(nbr,),
                                device_id_type=pl.DeviceIdType.MESH)
        pl.semaphore_wait(barrier, 2)

        def row_off(c, p):
            return c * (ch * N_PASS) + p * ch

        n_hops = 2 * (N_DEV - 1)
        n_sends = N_PASS * n_hops
        for p in range(N_PASS):
            cp = pltpu.make_async_copy(
                y_hbm.at[pl.ds(row_off(me, p), ch), :], acc, dma_sem)
            cp.start()
            cp.wait()

            for k in range(n_hops):
                kg = p * n_hops + k
                slot = kg % 2
                is_rs = k < N_DEV - 1

                if kg >= 2:
                    pl.semaphore_wait(credit_sem, 1)

                rdma = pltpu.make_async_remote_copy(
                    src_ref=acc,
                    dst_ref=recv.at[slot],
                    send_sem=send_sems.at[slot],
                    recv_sem=recv_sems.at[slot],
                    device_id=(right,),
                    device_id_type=pl.DeviceIdType.MESH,
                )
                rdma.start()

                if is_rs:
                    c_in = lax.rem(me - (k + 1) + 2 * N_DEV, N_DEV)
                    cpl = pltpu.make_async_copy(
                        y_hbm.at[pl.ds(row_off(c_in, p), ch), :], loc,
                        dma_sem)
                    cpl.start()
                    cpl.wait()

                rdma.wait()

                if is_rs:
                    acc[...] = recv[slot] + loc[...]
                else:
                    acc[...] = recv[slot]

                if kg <= n_sends - 3:
                    pl.semaphore_signal(credit_sem, inc=1,
                                        device_id=(left,),
                                        device_id_type=pl.DeviceIdType.MESH)

                if k == N_DEV - 2:
                    c_own = lax.rem(me + 1, N_DEV)
                    st = pltpu.make_async_copy(
                        acc, out_hbm.at[pl.ds(row_off(c_own, p), ch), :],
                        dma_sem)
                    st.start()
                    st.wait()
                if not is_rs:
                    t = k - (N_DEV - 1)
                    c_recv = lax.rem(me - t + 2 * N_DEV, N_DEV)
                    st = pltpu.make_async_copy(
                        acc, out_hbm.at[pl.ds(row_off(c_recv, p), ch), :],
                        dma_sem)
                    st.start()
                    st.wait()

    return pl.pallas_call(
        body,
        out_shape=jax.ShapeDtypeStruct((m, n), jnp.float32),
        in_specs=[pl.BlockSpec(memory_space=pl.ANY)],
        out_specs=pl.BlockSpec(memory_space=pl.ANY),
        scratch_shapes=[
            pltpu.VMEM((ch, n), jnp.float32),
            pltpu.VMEM((2, ch, n), jnp.float32),
            pltpu.VMEM((ch, n), jnp.float32),
            pltpu.SemaphoreType.DMA((2,)),
            pltpu.SemaphoreType.DMA((2,)),
            pltpu.SemaphoreType.DMA,
            pltpu.SemaphoreType.REGULAR,
        ],
        compiler_params=pltpu.CompilerParams(collective_id=0),
    )(y)
